# Initial kernel scaffold; baseline (speedup 1.0000x reference)
import jax
import jax.numpy as jnp
from jax.experimental import pallas as pl


def kernel(x, conv_w, conv_b, conv_bn_gamma, conv_bn_beta, conv_bn_mean, conv_bn_var, d3_w, d3_b, d3_bn_gamma, d3_bn_beta, d3_bn_mean, d3_bn_var, d1_w, d1_b, d1_bn_gamma, d1_bn_beta, d1_bn_mean, d1_bn_var, dd3_w, dd3_b, dd3_bn_gamma, dd3_bn_beta, dd3_bn_mean, dd3_bn_var, dd5_w, dd5_b, dd5_bn_gamma, dd5_bn_beta, dd5_bn_mean, dd5_bn_var, c1_w, c1_b, c1_bn_gamma, c1_bn_beta, c1_bn_mean, c1_bn_var, out_bn_gamma, out_bn_beta, out_bn_mean, out_bn_var):
    raise NotImplementedError("write your pallas kernel here")


# speedup vs baseline: 1.2281x; 1.2281x over previous
import jax
import jax.numpy as jnp
from jax import lax
from jax.experimental import pallas as pl
from jax.experimental.pallas import tpu as pltpu

EPS = 1e-5
INV_SQRT2 = 0.7071067811865476
HALO = 12


def _fold_bn(conv_bias, gamma, beta, mean, var):
    scale = gamma / jnp.sqrt(var + EPS)
    bias = beta + (conv_bias - mean) * scale
    return scale, bias


def _fused_kernel(xp_ref, wA_ref, sbA_ref, w9_ref, sbB_ref, w3_ref, sb3_ref,
                  w1_ref, sb1_ref, sb2_ref, o_ref, *, H, W, C, Cin):
    PH, PW = H + 2 * HALO, W + 2 * HALO
    SH, SW = PH - 8, PW - 8
    f32 = jnp.float32

    x2 = xp_ref[0].reshape(PH * PW, Cin)
    yA = jnp.dot(x2, wA_ref[...], preferred_element_type=f32)
    yA = jnp.maximum(yA * sbA_ref[0:1, :] + sbA_ref[1:2, :], 0.0)
    y3 = yA.reshape(PH, PW, C)
    hh = lax.broadcasted_iota(jnp.int32, (PH, PW, 1), 0)
    ww = lax.broadcasted_iota(jnp.int32, (PH, PW, 1), 1)
    inner = (hh >= HALO) & (hh < HALO + H) & (ww >= HALO) & (ww < HALO + W)
    y3 = jnp.where(inner, y3, 0.0)
    resid = y3[HALO:HALO + H, HALO:HALO + W, :].reshape(H * W, C)
    y_bf = y3.astype(jnp.bfloat16)

    acc = jnp.zeros((SH * SW, C), f32)
    for ky in range(3):
        for kx in range(3):
            t = ky * 3 + kx
            tap = y_bf[3 + ky:3 + ky + SH, 3 + kx:3 + kx + SW, :]
            acc = acc + jnp.dot(tap.reshape(SH * SW, C),
                                w9_ref[t * C:(t + 1) * C],
                                preferred_element_type=f32)
    z = jnp.maximum(acc * sbB_ref[0:1, :] + sbB_ref[1:2, :], 0.0)
    z3 = z.reshape(SH, SW, C)
    h2 = lax.broadcasted_iota(jnp.int32, (SH, SW, 1), 0)
    w2 = lax.broadcasted_iota(jnp.int32, (SH, SW, 1), 1)
    inner2 = (h2 >= 8) & (h2 < 8 + H) & (w2 >= 8) & (w2 < 8 + W)
    xb = jnp.where(inner2, z3, 0.0).astype(jnp.bfloat16)

    acc2 = jnp.zeros((H * W, C), f32)
    for bi, dil in enumerate((1, 3, 5)):
        accb = jnp.zeros((H * W, C), f32)
        for ky in range(3):
            for kx in range(3):
                t = ky * 3 + kx
                y0 = 8 + (ky - 1) * dil
                x0 = 8 + (kx - 1) * dil
                tap = xb[y0:y0 + H, x0:x0 + W, :]
                accb = accb + jnp.dot(tap.reshape(H * W, C),
                                      w3_ref[bi, t * C:(t + 1) * C],
                                      preferred_element_type=f32)
        zb = jnp.maximum(accb * sb3_ref[2 * bi:2 * bi + 1, :]
                         + sb3_ref[2 * bi + 1:2 * bi + 2, :], 0.0)
        acc2 = acc2 + jnp.dot(zb.astype(jnp.bfloat16), w1_ref[bi],
                              preferred_element_type=f32)
    y = jnp.maximum(acc2 * sb1_ref[0:1, :] + sb1_ref[1:2, :], 0.0)
    y = y + resid
    y = y * sb2_ref[0:1, :] + sb2_ref[1:2, :]
    y = 0.5 * y * (1.0 + lax.erf(y * INV_SQRT2))
    o_ref[0] = y


def kernel(x, conv_w, conv_b, conv_bn_gamma, conv_bn_beta, conv_bn_mean,
           conv_bn_var, d3_w, d3_b, d3_bn_gamma, d3_bn_beta, d3_bn_mean,
           d3_bn_var, d1_w, d1_b, d1_bn_gamma, d1_bn_beta, d1_bn_mean,
           d1_bn_var, dd3_w, dd3_b, dd3_bn_gamma, dd3_bn_beta, dd3_bn_mean,
           dd3_bn_var, dd5_w, dd5_b, dd5_bn_gamma, dd5_bn_beta, dd5_bn_mean,
           dd5_bn_var, c1_w, c1_b, c1_bn_gamma, c1_bn_beta, c1_bn_mean,
           c1_bn_var, out_bn_gamma, out_bn_beta, out_bn_mean, out_bn_var):
    B, Cin, H, W = x.shape
    C = conv_b.shape[0]
    bf16 = jnp.bfloat16

    xp = jnp.pad(jnp.transpose(x, (0, 2, 3, 1)),
                 ((0, 0), (HALO, HALO), (HALO, HALO), (0, 0))).astype(bf16)

    sA, bA = _fold_bn(conv_b, conv_bn_gamma, conv_bn_beta, conv_bn_mean,
                      conv_bn_var)
    sB, bB = _fold_bn(d3_b, d3_bn_gamma, d3_bn_beta, d3_bn_mean, d3_bn_var)
    s1d, b1d = _fold_bn(d1_b, d1_bn_gamma, d1_bn_beta, d1_bn_mean, d1_bn_var)
    s3d, b3d = _fold_bn(dd3_b, dd3_bn_gamma, dd3_bn_beta, dd3_bn_mean,
                        dd3_bn_var)
    s5d, b5d = _fold_bn(dd5_b, dd5_bn_gamma, dd5_bn_beta, dd5_bn_mean,
                        dd5_bn_var)
    s1, b1 = _fold_bn(c1_b, c1_bn_gamma, c1_bn_beta, c1_bn_mean, c1_bn_var)
    s2 = out_bn_gamma / jnp.sqrt(out_bn_var + EPS)
    b2 = out_bn_beta - out_bn_mean * s2

    wA = conv_w.astype(bf16)
    sbA = jnp.stack([sA, bA])
    w9 = d3_w.reshape(9 * C, C).astype(bf16)
    sbB = jnp.stack([sB, bB])
    w3 = jnp.stack([d1_w.reshape(9 * C, C), dd3_w.reshape(9 * C, C),
                    dd5_w.reshape(9 * C, C)]).astype(bf16)
    sb3 = jnp.stack([s1d, b1d, s3d, b3d, s5d, b5d])
    w1 = c1_w.reshape(3, C, C).astype(bf16)
    sb1 = jnp.stack([s1, b1])
    sb2 = jnp.stack([s2, b2])

    PH, PW = H + 2 * HALO, W + 2 * HALO
    import functools
    kern = functools.partial(_fused_kernel, H=H, W=W, C=C, Cin=Cin)
    out = pl.pallas_call(
        kern,
        out_shape=jax.ShapeDtypeStruct((B, H * W, C), jnp.float32),
        grid=(B,),
        in_specs=[
            pl.BlockSpec((1, PH, PW, Cin), lambda b: (b, 0, 0, 0)),
            pl.BlockSpec((Cin, C), lambda b: (0, 0)),
            pl.BlockSpec((2, C), lambda b: (0, 0)),
            pl.BlockSpec((9 * C, C), lambda b: (0, 0)),
            pl.BlockSpec((2, C), lambda b: (0, 0)),
            pl.BlockSpec((3, 9 * C, C), lambda b: (0, 0, 0)),
            pl.BlockSpec((6, C), lambda b: (0, 0)),
            pl.BlockSpec((3, C, C), lambda b: (0, 0, 0)),
            pl.BlockSpec((2, C), lambda b: (0, 0)),
            pl.BlockSpec((2, C), lambda b: (0, 0)),
        ],
        out_specs=pl.BlockSpec((1, H * W, C), lambda b: (b, 0, 0)),
        compiler_params=pltpu.CompilerParams(
            dimension_semantics=("parallel",),
            vmem_limit_bytes=60 * 1024 * 1024),
    )(xp, wA, sbA, w9, sbB, w3, sb3, w1, sb1, sb2)

    return jnp.transpose(out.reshape(B, H, W, C), (0, 3, 1, 2))


# speedup vs baseline: 2.0396x; 1.6608x over previous
import functools

import jax
import jax.numpy as jnp
from jax import lax
from jax.experimental import pallas as pl
from jax.experimental.pallas import tpu as pltpu

EPS = 1e-5
INV_SQRT2 = 0.7071067811865476
HALO = 12
HC_STEM = 16
HC_TAIL = 16


def _fold_bn(conv_bias, gamma, beta, mean, var):
    scale = gamma / jnp.sqrt(var + EPS)
    bias = beta + (conv_bias - mean) * scale
    return scale, bias


def _im2col_dot(src, row0, col0, hc, wc, offs, w_ref, C):
    taps = []
    for (dy, dx) in offs:
        t = src[row0 + dy:row0 + dy + hc, col0 + dx:col0 + dx + wc, :]
        taps.append(t.reshape(hc * wc, C))
    xcol = jnp.concatenate(taps, axis=-1)
    return jnp.dot(xcol, w_ref, preferred_element_type=jnp.float32)


def _fused_kernel(xp_ref, wA_ref, sbA_ref, w9_ref, sbB_ref, w3_ref, sb3_ref,
                  w1_ref, sb1_ref, sb2_ref, o_ref, *, H, W, C, Cin):
    PH, PW = H + 2 * HALO, W + 2 * HALO
    SH, SW = PH - 8, PW - 8
    f32 = jnp.float32
    bf16 = jnp.bfloat16
    offs3 = [(ky, kx) for ky in (-1, 0, 1) for kx in (-1, 0, 1)]

    x2 = xp_ref[0].reshape(PH * PW, Cin)
    yA = jnp.dot(x2, wA_ref[...], preferred_element_type=f32)
    yA = jnp.maximum(yA * sbA_ref[0:1, :] + sbA_ref[1:2, :], 0.0)
    y3 = yA.reshape(PH, PW, C)
    hh = lax.broadcasted_iota(jnp.int32, (PH, PW, 1), 0)
    ww = lax.broadcasted_iota(jnp.int32, (PH, PW, 1), 1)
    inner = (hh >= HALO) & (hh < HALO + H) & (ww >= HALO) & (ww < HALO + W)
    y3 = jnp.where(inner, y3, 0.0)
    resid = y3[HALO:HALO + H, HALO:HALO + W, :].reshape(H * W, C)
    y_bf = y3.astype(bf16)

    ww2 = lax.broadcasted_iota(jnp.int32, (HC_STEM, SW, 1), 1)
    chunks = []
    for h0 in range(0, SH, HC_STEM):
        z = _im2col_dot(y_bf, h0 + 4, 4, HC_STEM, SW, offs3, w9_ref[...], C)
        z = jnp.maximum(z * sbB_ref[0:1, :] + sbB_ref[1:2, :], 0.0)
        z = z.reshape(HC_STEM, SW, C)
        hh2 = lax.broadcasted_iota(jnp.int32, (HC_STEM, SW, 1), 0) + h0
        good = (hh2 >= 8) & (hh2 < 8 + H) & (ww2 >= 8) & (ww2 < 8 + W)
        chunks.append(jnp.where(good, z, 0.0).astype(bf16))
    xb = jnp.concatenate(chunks, axis=0)

    for i0 in range(0, H, HC_TAIL):
        acc = jnp.zeros((HC_TAIL * W, C), f32)
        for bi, dil in enumerate((1, 3, 5)):
            offs = [(ky * dil, kx * dil) for ky in (-1, 0, 1)
                    for kx in (-1, 0, 1)]
            zb = _im2col_dot(xb, i0 + 8, 8, HC_TAIL, W, offs, w3_ref[bi], C)
            zb = jnp.maximum(zb * sb3_ref[2 * bi:2 * bi + 1, :]
                             + sb3_ref[2 * bi + 1:2 * bi + 2, :], 0.0)
            acc = acc + jnp.dot(zb.astype(bf16), w1_ref[bi],
                                preferred_element_type=f32)
        y = jnp.maximum(acc * sb1_ref[0:1, :] + sb1_ref[1:2, :], 0.0)
        y = y + resid[i0 * W:(i0 + HC_TAIL) * W, :]
        y = y * sb2_ref[0:1, :] + sb2_ref[1:2, :]
        y = 0.5 * y * (1.0 + lax.erf(y * INV_SQRT2))
        o_ref[0, i0 * W:(i0 + HC_TAIL) * W, :] = y


def kernel(x, conv_w, conv_b, conv_bn_gamma, conv_bn_beta, conv_bn_mean,
           conv_bn_var, d3_w, d3_b, d3_bn_gamma, d3_bn_beta, d3_bn_mean,
           d3_bn_var, d1_w, d1_b, d1_bn_gamma, d1_bn_beta, d1_bn_mean,
           d1_bn_var, dd3_w, dd3_b, dd3_bn_gamma, dd3_bn_beta, dd3_bn_mean,
           dd3_bn_var, dd5_w, dd5_b, dd5_bn_gamma, dd5_bn_beta, dd5_bn_mean,
           dd5_bn_var, c1_w, c1_b, c1_bn_gamma, c1_bn_beta, c1_bn_mean,
           c1_bn_var, out_bn_gamma, out_bn_beta, out_bn_mean, out_bn_var):
    B, Cin, H, W = x.shape
    C = conv_b.shape[0]
    bf16 = jnp.bfloat16

    xp = jnp.pad(jnp.transpose(x, (0, 2, 3, 1)),
                 ((0, 0), (HALO, HALO), (HALO, HALO), (0, 0))).astype(bf16)

    sA, bA = _fold_bn(conv_b, conv_bn_gamma, conv_bn_beta, conv_bn_mean,
                      conv_bn_var)
    sB, bB = _fold_bn(d3_b, d3_bn_gamma, d3_bn_beta, d3_bn_mean, d3_bn_var)
    s1d, b1d = _fold_bn(d1_b, d1_bn_gamma, d1_bn_beta, d1_bn_mean, d1_bn_var)
    s3d, b3d = _fold_bn(dd3_b, dd3_bn_gamma, dd3_bn_beta, dd3_bn_mean,
                        dd3_bn_var)
    s5d, b5d = _fold_bn(dd5_b, dd5_bn_gamma, dd5_bn_beta, dd5_bn_mean,
                        dd5_bn_var)
    s1, b1 = _fold_bn(c1_b, c1_bn_gamma, c1_bn_beta, c1_bn_mean, c1_bn_var)
    s2 = out_bn_gamma / jnp.sqrt(out_bn_var + EPS)
    b2 = out_bn_beta - out_bn_mean * s2

    wA = conv_w.astype(bf16)
    sbA = jnp.stack([sA, bA])
    w9 = d3_w.reshape(9 * C, C).astype(bf16)
    sbB = jnp.stack([sB, bB])
    w3 = jnp.stack([d1_w.reshape(9 * C, C), dd3_w.reshape(9 * C, C),
                    dd5_w.reshape(9 * C, C)]).astype(bf16)
    sb3 = jnp.stack([s1d, b1d, s3d, b3d, s5d, b5d])
    w1 = c1_w.reshape(3, C, C).astype(bf16)
    sb1 = jnp.stack([s1, b1])
    sb2 = jnp.stack([s2, b2])

    PH, PW = H + 2 * HALO, W + 2 * HALO
    kern = functools.partial(_fused_kernel, H=H, W=W, C=C, Cin=Cin)
    out = pl.pallas_call(
        kern,
        out_shape=jax.ShapeDtypeStruct((B, H * W, C), jnp.float32),
        grid=(B,),
        in_specs=[
            pl.BlockSpec((1, PH, PW, Cin), lambda b: (b, 0, 0, 0)),
            pl.BlockSpec((Cin, C), lambda b: (0, 0)),
            pl.BlockSpec((2, C), lambda b: (0, 0)),
            pl.BlockSpec((9 * C, C), lambda b: (0, 0)),
            pl.BlockSpec((2, C), lambda b: (0, 0)),
            pl.BlockSpec((3, 9 * C, C), lambda b: (0, 0, 0)),
            pl.BlockSpec((6, C), lambda b: (0, 0)),
            pl.BlockSpec((3, C, C), lambda b: (0, 0, 0)),
            pl.BlockSpec((2, C), lambda b: (0, 0)),
            pl.BlockSpec((2, C), lambda b: (0, 0)),
        ],
        out_specs=pl.BlockSpec((1, H * W, C), lambda b: (b, 0, 0)),
        compiler_params=pltpu.CompilerParams(
            dimension_semantics=("parallel",),
            vmem_limit_bytes=60 * 1024 * 1024),
    )(xp, wA, sbA, w9, sbB, w3, sb3, w1, sb1, sb2)

    return jnp.transpose(out.reshape(B, H, W, C), (0, 3, 1, 2))
